# baseline (speedup 1.0000x reference)
import functools

import jax
import jax.numpy as jnp
from jax import lax
from jax.experimental import pallas as pl
from jax.experimental.pallas import tpu as pltpu
from jax.experimental.pallas import tpu_sc as plsc

N = 10000
E = 320000
HID = 128
OUT = 64
L_PE = 8
N_PERM = 8

NPAD = 10240
EPT = 10240
E_PAD = 32 * EPT
K = 128
CHUNKS = EPT // K
ROWS_T = NPAD // 16

BN = 1280
BNF = 1000

_mesh = plsc.VectorSubcoreMesh(core_axis_name="c", subcore_axis_name="s")


def _agg_body(with_gather, *refs):
    if with_gather:
        (src_hbm, dst_hbm, h_hbm, zeros_hbm,
         out_hbm, acc_s, idx_v, dstv, rows_v, sem) = refs
    else:
        (dst_hbm, zeros_hbm, ones_hbm,
         out_hbm, acc_s, dstv, rows_v, sem) = refs
    ci = lax.axis_index("c")
    sid = lax.axis_index("s")
    base = (ci * 16 + sid) * EPT

    pltpu.sync_copy(zeros_hbm, acc_s.at[pl.ds(sid * ROWS_T, ROWS_T)])
    if not with_gather:
        pltpu.sync_copy(ones_hbm, rows_v)
    plsc.subcore_barrier()

    def chunk(i, carry):
        off = base + i * K
        pltpu.sync_copy(dst_hbm.at[pl.ds(off, K)], dstv)
        if with_gather:
            pltpu.sync_copy(src_hbm.at[pl.ds(off, K)], idx_v)
            pltpu.async_copy(h_hbm.at[idx_v], rows_v, sem).wait()
        pltpu.sync_copy(rows_v, acc_s.at[dstv], add=True)
        return carry

    lax.fori_loop(0, CHUNKS, chunk, 0)
    plsc.subcore_barrier()

    sl = pl.ds(sid * ROWS_T, ROWS_T)
    pltpu.sync_copy(acc_s.at[sl], out_hbm.at[ci, sl])


def _make_agg_call():
    return pl.kernel(
        functools.partial(_agg_body, True),
        mesh=_mesh,
        out_type=jax.ShapeDtypeStruct((2, NPAD, HID), jnp.float32),
        scratch_types=[
            pltpu.VMEM_SHARED((NPAD, HID), jnp.float32),
            pltpu.VMEM((K,), jnp.int32),
            pltpu.VMEM((K,), jnp.int32),
            pltpu.VMEM((K, HID), jnp.float32),
            pltpu.SemaphoreType.DMA,
        ],
    )


_agg_calls = [_make_agg_call() for _ in range(3)]

_deg_call = pl.kernel(
    functools.partial(_agg_body, False),
    mesh=_mesh,
    out_type=jax.ShapeDtypeStruct((2, NPAD, HID), jnp.float32),
    scratch_types=[
        pltpu.VMEM_SHARED((NPAD, HID), jnp.float32),
        pltpu.VMEM((K,), jnp.int32),
        pltpu.VMEM((K, HID), jnp.float32),
        pltpu.SemaphoreType.DMA,
    ],
)


def _gcn_mm_body(a0_ref, a1_ref, d0_ref, d1_ref, w_ref, b_ref, o_ref):
    i = pl.program_id(0)
    a = a0_ref[0] + a1_ref[0]
    d = d0_ref[0][:, 0:1] + d1_ref[0][:, 0:1]
    a = a / jnp.maximum(d, 1.0)
    y = jax.nn.relu(
        jnp.dot(a, w_ref[...], preferred_element_type=jnp.float32) + b_ref[...]
    )
    rowid = i * BN + lax.broadcasted_iota(jnp.int32, (BN, 1), 0)
    o_ref[...] = jnp.where(rowid < N, y, 0.0)


def _gcn_mm(agg, deg, w, b):
    return pl.pallas_call(
        _gcn_mm_body,
        grid=(NPAD // BN,),
        in_specs=[
            pl.BlockSpec((1, BN, HID), lambda i: (0, i, 0)),
            pl.BlockSpec((1, BN, HID), lambda i: (1, i, 0)),
            pl.BlockSpec((1, BN, HID), lambda i: (0, i, 0)),
            pl.BlockSpec((1, BN, HID), lambda i: (1, i, 0)),
            pl.BlockSpec((HID, HID), lambda i: (0, 0)),
            pl.BlockSpec((1, HID), lambda i: (0, 0)),
        ],
        out_specs=pl.BlockSpec((BN, HID), lambda i: (i, 0)),
        out_shape=jax.ShapeDtypeStruct((NPAD, HID), jnp.float32),
    )(agg, agg, deg, deg, w, b.reshape(1, HID))


def _mm_stats_body(a_ref, w_ref, b_ref, o_ref, s_ref, ss_ref):
    i = pl.program_id(0)
    y = jnp.dot(a_ref[...], w_ref[...], preferred_element_type=jnp.float32) + b_ref[...]
    o_ref[...] = y

    @pl.when(i == 0)
    def _init():
        s_ref[...] = jnp.zeros_like(s_ref)
        ss_ref[...] = jnp.zeros_like(ss_ref)

    s_ref[...] += jnp.sum(y, axis=0, keepdims=True)
    ss_ref[...] += jnp.sum(y * y, axis=0, keepdims=True)


def _mm_stats(a, w, b):
    n, k = a.shape
    m = w.shape[1]
    return pl.pallas_call(
        _mm_stats_body,
        grid=(n // BNF,),
        in_specs=[
            pl.BlockSpec((BNF, k), lambda i: (i, 0)),
            pl.BlockSpec((k, m), lambda i: (0, 0)),
            pl.BlockSpec((1, m), lambda i: (0, 0)),
        ],
        out_specs=[
            pl.BlockSpec((BNF, m), lambda i: (i, 0)),
            pl.BlockSpec((1, m), lambda i: (0, 0)),
            pl.BlockSpec((1, m), lambda i: (0, 0)),
        ],
        out_shape=[
            jax.ShapeDtypeStruct((n, m), jnp.float32),
            jax.ShapeDtypeStruct((1, m), jnp.float32),
            jax.ShapeDtypeStruct((1, m), jnp.float32),
        ],
    )(a, w, b.reshape(1, m))


def _bn_relu_mm_body(y_ref, sc_ref, sh_ref, w_ref, b_ref, o_ref):
    h = jax.nn.relu(y_ref[...] * sc_ref[...] + sh_ref[...])
    o_ref[...] = (
        jnp.dot(h, w_ref[...], preferred_element_type=jnp.float32) + b_ref[...]
    )


def _bn_relu_mm(y, scale, shift, w, b):
    n, k = y.shape
    m = w.shape[1]
    return pl.pallas_call(
        _bn_relu_mm_body,
        grid=(n // BNF,),
        in_specs=[
            pl.BlockSpec((BNF, k), lambda i: (i, 0)),
            pl.BlockSpec((1, k), lambda i: (0, 0)),
            pl.BlockSpec((1, k), lambda i: (0, 0)),
            pl.BlockSpec((k, m), lambda i: (0, 0)),
            pl.BlockSpec((1, m), lambda i: (0, 0)),
        ],
        out_specs=pl.BlockSpec((BNF, m), lambda i: (i, 0)),
        out_shape=jax.ShapeDtypeStruct((n, m), jnp.float32),
    )(y, scale.reshape(1, k), shift.reshape(1, k), w, b.reshape(1, m))


def _bn_affine(s, ss, n, g, beta):
    mu = s[0] / n
    var = ss[0] / n - mu * mu
    inv = g / jnp.sqrt(var + 1e-5)
    return inv, beta - mu * inv


def kernel(x, edge_index, idx, W1, b1, W2, b2, W3, b3, Wf1, bf1, gf1, betaf1,
           Wf2, bf2, Wp1, bp1, gp1, betap1, Wp2, bp2, perm_table):
    src, dst = edge_index[0], edge_index[1]
    pad = jnp.full((E_PAD - E,), N, jnp.int32)
    src_p = jnp.concatenate([src, pad])
    dst_p = jnp.concatenate([dst, pad])

    x_pad = jnp.pad(x, ((0, NPAD - N), (0, 0)))
    zeros = jnp.zeros((ROWS_T, HID), jnp.float32)
    ones = jnp.ones((K, HID), jnp.float32)

    deg = _deg_call(dst_p, zeros, ones)
    x_pad = lax.optimization_barrier((x_pad, deg))[0]

    agg = _agg_calls[0](src_p, dst_p, x_pad, zeros)
    h = _gcn_mm(agg, deg, W1, b1)
    agg = _agg_calls[1](src_p, dst_p, h, zeros)
    h = _gcn_mm(agg, deg, W2, b2)
    agg = _agg_calls[2](src_p, dst_p, h, zeros)
    xs = _gcn_mm(agg, deg, W3, b3)[:N]

    y1, s1, ss1 = _mm_stats(xs, Wf1, bf1)
    sc1, sh1 = _bn_affine(s1, ss1, N, gf1, betaf1)
    regions = _bn_relu_mm(y1, sc1, sh1, Wf2, bf2)

    pt = lax.optimization_barrier((perm_table, xs))[0]
    c = pt[idx]
    coors = [c]
    for _ in range(L_PE - 1):
        m = jax.ops.segment_max(c[src], dst, num_segments=N)
        c = jnp.maximum(c, m)
        coors.append(c)
    trans = jnp.stack(coors, axis=0).transpose(1, 2, 0).reshape(N, N_PERM * L_PE)

    y2, s2, ss2 = _mm_stats(trans, Wp1, bp1)
    sc2, sh2 = _bn_affine(s2, ss2, N, gp1, betap1)
    pe = _bn_relu_mm(y2, sc2, sh2, Wp2, bp2)
    return (regions, pe)


# speedup vs baseline: 1.1054x; 1.1054x over previous
import jax
import jax.numpy as jnp
from jax import lax
from jax.experimental import pallas as pl
from jax.experimental.pallas import tpu as pltpu
from jax.experimental.pallas import tpu_sc as plsc

N = 10000
E = 320000
HID = 128
OUT = 64
L_PE = 8
N_PERM = 8

NPAD = 10240
BPT = NPAD // 32
GK = 64

BNF = 1000

_mesh = plsc.VectorSubcoreMesh(core_axis_name="c", subcore_axis_name="s")


def _gather_body(tab_hbm, idx_hbm, out_hbm, idx_v, rows_v, sem):
    ci = lax.axis_index("c")
    sid = lax.axis_index("s")
    base = (ci * 16 + sid) * BPT

    def chunk(i, carry):
        off = base + i * GK
        pltpu.sync_copy(idx_hbm.at[pl.ds(off, GK)], idx_v)
        pltpu.async_copy(tab_hbm.at[idx_v], rows_v, sem).wait()
        pltpu.sync_copy(rows_v, out_hbm.at[pl.ds(off, GK)])
        return carry

    lax.fori_loop(0, BPT // GK, chunk, 0)


_gather_call = pl.kernel(
    _gather_body,
    mesh=_mesh,
    out_type=jax.ShapeDtypeStruct((NPAD, 128), jnp.float32),
    scratch_types=[
        pltpu.VMEM((GK,), jnp.int32),
        pltpu.VMEM((GK, 128), jnp.float32),
        pltpu.SemaphoreType.DMA,
    ],
)


def _gcn_mm_body(a_ref, d_ref, w_ref, b_ref, o_ref):
    a = a_ref[...] / jnp.maximum(d_ref[...], 1.0)
    o_ref[...] = jax.nn.relu(
        jnp.dot(a, w_ref[...], preferred_element_type=jnp.float32) + b_ref[...]
    )


def _gcn_mm(agg, deg, w, b):
    return pl.pallas_call(
        _gcn_mm_body,
        grid=(N // BNF,),
        in_specs=[
            pl.BlockSpec((BNF, HID), lambda i: (i, 0)),
            pl.BlockSpec((BNF, 1), lambda i: (i, 0)),
            pl.BlockSpec((HID, HID), lambda i: (0, 0)),
            pl.BlockSpec((1, HID), lambda i: (0, 0)),
        ],
        out_specs=pl.BlockSpec((BNF, HID), lambda i: (i, 0)),
        out_shape=jax.ShapeDtypeStruct((N, HID), jnp.float32),
    )(agg, deg.reshape(N, 1), w, b.reshape(1, HID))


def _mm_stats_body(a_ref, w_ref, b_ref, o_ref, s_ref, ss_ref):
    i = pl.program_id(0)
    y = jnp.dot(a_ref[...], w_ref[...], preferred_element_type=jnp.float32) + b_ref[...]
    o_ref[...] = y

    @pl.when(i == 0)
    def _init():
        s_ref[...] = jnp.zeros_like(s_ref)
        ss_ref[...] = jnp.zeros_like(ss_ref)

    s_ref[...] += jnp.sum(y, axis=0, keepdims=True)
    ss_ref[...] += jnp.sum(y * y, axis=0, keepdims=True)


def _mm_stats(a, w, b):
    n, k = a.shape
    m = w.shape[1]
    return pl.pallas_call(
        _mm_stats_body,
        grid=(n // BNF,),
        in_specs=[
            pl.BlockSpec((BNF, k), lambda i: (i, 0)),
            pl.BlockSpec((k, m), lambda i: (0, 0)),
            pl.BlockSpec((1, m), lambda i: (0, 0)),
        ],
        out_specs=[
            pl.BlockSpec((BNF, m), lambda i: (i, 0)),
            pl.BlockSpec((1, m), lambda i: (0, 0)),
            pl.BlockSpec((1, m), lambda i: (0, 0)),
        ],
        out_shape=[
            jax.ShapeDtypeStruct((n, m), jnp.float32),
            jax.ShapeDtypeStruct((1, m), jnp.float32),
            jax.ShapeDtypeStruct((1, m), jnp.float32),
        ],
    )(a, w, b.reshape(1, m))


def _bn_relu_mm_body(y_ref, sc_ref, sh_ref, w_ref, b_ref, o_ref):
    h = jax.nn.relu(y_ref[...] * sc_ref[...] + sh_ref[...])
    o_ref[...] = (
        jnp.dot(h, w_ref[...], preferred_element_type=jnp.float32) + b_ref[...]
    )


def _bn_relu_mm(y, scale, shift, w, b):
    n, k = y.shape
    m = w.shape[1]
    return pl.pallas_call(
        _bn_relu_mm_body,
        grid=(n // BNF,),
        in_specs=[
            pl.BlockSpec((BNF, k), lambda i: (i, 0)),
            pl.BlockSpec((1, k), lambda i: (0, 0)),
            pl.BlockSpec((1, k), lambda i: (0, 0)),
            pl.BlockSpec((k, m), lambda i: (0, 0)),
            pl.BlockSpec((1, m), lambda i: (0, 0)),
        ],
        out_specs=pl.BlockSpec((BNF, m), lambda i: (i, 0)),
        out_shape=jax.ShapeDtypeStruct((n, m), jnp.float32),
    )(y, scale.reshape(1, k), shift.reshape(1, k), w, b.reshape(1, m))


def _bn_affine(s, ss, n, g, beta):
    mu = s[0] / n
    var = ss[0] / n - mu * mu
    inv = g / jnp.sqrt(var + 1e-5)
    return inv, beta - mu * inv


def kernel(x, edge_index, idx, W1, b1, W2, b2, W3, b3, Wf1, bf1, gf1, betaf1,
           Wf2, bf2, Wp1, bp1, gp1, betap1, Wp2, bp2, perm_table):
    n = x.shape[0]
    src, dst = edge_index[0], edge_index[1]

    deg = jax.ops.segment_sum(jnp.ones((E,), jnp.float32), dst, num_segments=n)

    h = x
    for (W, b) in ((W1, b1), (W2, b2), (W3, b3)):
        agg = jax.ops.segment_sum(h[src], dst, num_segments=n)
        h = _gcn_mm(agg, deg, W, b)
    xs = h

    y1, s1, ss1 = _mm_stats(xs, Wf1, bf1)
    sc1, sh1 = _bn_affine(s1, ss1, n, gf1, betaf1)
    regions = _bn_relu_mm(y1, sc1, sh1, Wf2, bf2)

    pt128 = jnp.pad(perm_table, ((0, 0), (0, 128 - N_PERM)))
    idx_pad = jnp.pad(idx, (0, NPAD - N))
    c = _gather_call(pt128, idx_pad)[:N, :N_PERM]

    coors = [c]
    for _ in range(L_PE - 1):
        m = jax.ops.segment_max(c[src], dst, num_segments=n)
        c = jnp.maximum(c, m)
        coors.append(c)
    trans = jnp.stack(coors, axis=0).transpose(1, 2, 0).reshape(n, N_PERM * L_PE)

    y2, s2, ss2 = _mm_stats(trans, Wp1, bp1)
    sc2, sh2 = _bn_affine(s2, ss2, n, gp1, betap1)
    pe = _bn_relu_mm(y2, sc2, sh2, Wp2, bp2)
    return (regions, pe)
